# Initial kernel scaffold; baseline (speedup 1.0000x reference)
import jax
import jax.numpy as jnp
from jax.experimental import pallas as pl


def kernel(x, edge_index, edge_attr, batch, c1_We, c1_be, c1_W1, c1_b1, c1_g, c1_bt, c1_W2, c1_b2, c2_We, c2_be, c2_W1, c2_b1, c2_g, c2_bt, c2_W2, c2_b2, c3_We, c3_be, c3_W1, c3_b1, c3_g, c3_bt, c3_W2, c3_b2, W_l1, b_l1, W_l2, b_l2):
    raise NotImplementedError("write your pallas kernel here")


# speedup vs baseline: 2.5870x; 2.5870x over previous
import functools

import jax
import jax.numpy as jnp
from jax import lax
from jax.experimental import pallas as pl
from jax.experimental.pallas import tpu as pltpu
from jax.experimental.pallas import tpu_sc as plsc

N, E, D, DE, H, G = 10000, 320000, 128, 16, 128, 64

_NC, _NS = 2, 16
_NW = _NC * _NS
_EPW = E // _NW
_C = 80
_NCHUNK = _EPW // _C
_NPAD = 10240
_RPT = _NPAD // _NS
_ZR = 128

_BN_INV = 1.0 / (1.0 + 1e-5) ** 0.5


_sc_mesh = plsc.VectorSubcoreMesh(core_axis_name="c", subcore_axis_name="s")


@functools.partial(
    pl.kernel,
    out_type=jax.ShapeDtypeStruct((_NC * _NPAD, D), jnp.float32),
    mesh=_sc_mesh,
    scratch_types=[
        pltpu.VMEM((_C,), jnp.int32),
        pltpu.VMEM((_C, D), jnp.float32),
        pltpu.VMEM((_ZR, D), jnp.float32),
        pltpu.VMEM_SHARED((_NPAD, D), jnp.float32),
        pltpu.SemaphoreType.DMA,
    ],
)
def _sc_agg(x_hbm, e_hbm, src_hbm, dst_hbm, out_hbm,
            idx_v, buf_v, zbuf_v, agg_sh, sem):
    c = lax.axis_index("c")
    s = lax.axis_index("s")
    w = c * _NS + s

    def _zrow(j, carry):
        for k in range(D // 16):
            zbuf_v[j, pl.ds(k * 16, 16)] = jnp.zeros((16,), jnp.float32)
        return carry

    lax.fori_loop(0, _ZR, _zrow, 0)
    for k in range(_RPT // _ZR):
        pltpu.sync_copy(zbuf_v, agg_sh.at[pl.ds(s * _RPT + k * _ZR, _ZR)])
    plsc.subcore_barrier()

    def _chunk(i, carry):
        base = w * _EPW + i * _C
        pltpu.sync_copy(src_hbm.at[pl.ds(base, _C)], idx_v)
        pltpu.sync_copy(e_hbm.at[pl.ds(base, _C)], buf_v)
        pltpu.async_copy(x_hbm.at[idx_v], buf_v, sem, add=True).wait()

        def _rrow(j, cc):
            for k in range(D // 16):
                v = buf_v[j, pl.ds(k * 16, 16)]
                buf_v[j, pl.ds(k * 16, 16)] = jnp.maximum(v, 0.0)
            return cc

        lax.fori_loop(0, _C, _rrow, 0)
        pltpu.sync_copy(dst_hbm.at[pl.ds(base, _C)], idx_v)
        pltpu.sync_copy(buf_v, agg_sh.at[idx_v], add=True)
        return carry

    lax.fori_loop(0, _NCHUNK, _chunk, 0)
    plsc.subcore_barrier()

    for k in range(_RPT // _ZR):
        r = s * _RPT + k * _ZR
        pltpu.sync_copy(agg_sh.at[pl.ds(r, _ZR)],
                        out_hbm.at[pl.ds(c * _NPAD + r, _ZR)])


_BE = 4000


def _edge_mm_body(ea_ref, We_ref, be_ref, out_ref):
    out_ref[...] = (
        jnp.dot(ea_ref[...], We_ref[...], preferred_element_type=jnp.float32)
        + be_ref[...]
    )


def _edge_mm(ea, We, be):
    return pl.pallas_call(
        _edge_mm_body,
        grid=(E // _BE,),
        in_specs=[
            pl.BlockSpec((_BE, DE), lambda i: (i, 0)),
            pl.BlockSpec((DE, H), lambda i: (0, 0)),
            pl.BlockSpec((1, H), lambda i: (0, 0)),
        ],
        out_specs=pl.BlockSpec((_BE, H), lambda i: (i, 0)),
        out_shape=jax.ShapeDtypeStruct((E, H), jnp.float32),
    )(ea, We, be.reshape(1, H))


_BNODE = 1000


def _layer_body(x_ref, a0_ref, a1_ref, b_ref, W1_ref, b1_ref, g_ref, bt_ref,
                W2_ref, b2_ref, h_ref, ps_ref, cnt_ref=None):
    i = pl.program_id(0)
    h = x_ref[...] + a0_ref[...] + a1_ref[...]
    t = jnp.dot(h, W1_ref[...], preferred_element_type=jnp.float32) + b1_ref[...]
    t = t * (_BN_INV * g_ref[...]) + bt_ref[...]
    t = jnp.maximum(t, 0.0)
    u = jnp.dot(t, W2_ref[...], preferred_element_type=jnp.float32) + b2_ref[...]
    u = jnp.maximum(u, 0.0)
    h_ref[...] = u
    onehot = (b_ref[...] == lax.broadcasted_iota(jnp.int32, (1, G), 1)
              ).astype(jnp.float32)
    ps = lax.dot_general(onehot, u, (((0,), (0,)), ((), ())),
                         preferred_element_type=jnp.float32)

    @pl.when(i == 0)
    def _():
        ps_ref[...] = ps

    @pl.when(i > 0)
    def _():
        ps_ref[...] += ps

    if cnt_ref is not None:
        cm = lax.dot_general(
            onehot, jnp.ones((_BNODE, H), jnp.float32),
            (((0,), (0,)), ((), ())), preferred_element_type=jnp.float32)

        @pl.when(i == 0)
        def _():
            cnt_ref[...] = cm

        @pl.when(i > 0)
        def _():
            cnt_ref[...] += cm


def _layer(x, a0, a1, batch2, W1, b1, g, bt, W2, b2, with_counts):
    body = _layer_body if with_counts else (
        lambda *refs: _layer_body(*refs, cnt_ref=None))
    row = lambda v: v.reshape(1, H)
    out_shape = [
        jax.ShapeDtypeStruct((N, H), jnp.float32),
        jax.ShapeDtypeStruct((G, H), jnp.float32),
    ]
    out_specs = [
        pl.BlockSpec((_BNODE, H), lambda i: (i, 0)),
        pl.BlockSpec((G, H), lambda i: (0, 0)),
    ]
    if with_counts:
        out_shape.append(jax.ShapeDtypeStruct((G, H), jnp.float32))
        out_specs.append(pl.BlockSpec((G, H), lambda i: (0, 0)))
    return pl.pallas_call(
        body,
        grid=(N // _BNODE,),
        in_specs=[
            pl.BlockSpec((_BNODE, H), lambda i: (i, 0)),
            pl.BlockSpec((_BNODE, H), lambda i: (i, 0)),
            pl.BlockSpec((_BNODE, H), lambda i: (i, 0)),
            pl.BlockSpec((_BNODE, 1), lambda i: (i, 0)),
            pl.BlockSpec((H, H), lambda i: (0, 0)),
            pl.BlockSpec((1, H), lambda i: (0, 0)),
            pl.BlockSpec((1, H), lambda i: (0, 0)),
            pl.BlockSpec((1, H), lambda i: (0, 0)),
            pl.BlockSpec((H, H), lambda i: (0, 0)),
            pl.BlockSpec((1, H), lambda i: (0, 0)),
        ],
        out_specs=out_specs,
        out_shape=out_shape,
    )(x, a0, a1, batch2, W1, row(b1), row(g), row(bt), W2, row(b2))


def _head_body(p1_ref, p2_ref, p3_ref, cnt_ref, W1_ref, b1_ref, W2_ref,
               b2_ref, out_ref):
    c = jnp.maximum(cnt_ref[...][:, 0:1], 1.0)
    q = jnp.concatenate(
        [p1_ref[...] / c, p2_ref[...] / c, p3_ref[...] / c], axis=1)
    t = jnp.dot(q, W1_ref[...], preferred_element_type=jnp.float32) + b1_ref[...]
    t = jnp.maximum(t, 0.0)
    z = jnp.dot(t, W2_ref[...], preferred_element_type=jnp.float32) + b2_ref[...]
    out_ref[...] = jax.nn.sigmoid(z)


def _head(p1, p2, p3, cnt, W1, b1, W2, b2):
    return pl.pallas_call(
        _head_body,
        out_shape=jax.ShapeDtypeStruct((G, 1), jnp.float32),
    )(p1, p2, p3, cnt, W1, b1.reshape(1, -1), W2, b2.reshape(1, -1))


def kernel(x, edge_index, edge_attr, batch,
           c1_We, c1_be, c1_W1, c1_b1, c1_g, c1_bt, c1_W2, c1_b2,
           c2_We, c2_be, c2_W1, c2_b1, c2_g, c2_bt, c2_W2, c2_b2,
           c3_We, c3_be, c3_W1, c3_b1, c3_g, c3_bt, c3_W2, c3_b2,
           W_l1, b_l1, W_l2, b_l2):
    src = edge_index[0]
    dst = edge_index[1]
    batch2 = batch.reshape(N, 1)

    e1 = _edge_mm(edge_attr, c1_We, c1_be)
    e2 = _edge_mm(edge_attr, c2_We, c2_be)
    e3 = _edge_mm(edge_attr, c3_We, c3_be)

    a1 = _sc_agg(x, e1, src, dst)
    h1, p1, cnt = _layer(x, a1[:N], a1[_NPAD:_NPAD + N], batch2,
                         c1_W1, c1_b1, c1_g, c1_bt, c1_W2, c1_b2, True)
    a2 = _sc_agg(h1, e2, src, dst)
    h2, p2 = _layer(h1, a2[:N], a2[_NPAD:_NPAD + N], batch2,
                    c2_W1, c2_b1, c2_g, c2_bt, c2_W2, c2_b2, False)
    a3 = _sc_agg(h2, e3, src, dst)
    h3, p3 = _layer(h2, a3[:N], a3[_NPAD:_NPAD + N], batch2,
                    c3_W1, c3_b1, c3_g, c3_bt, c3_W2, c3_b2, False)

    return _head(p1, p2, p3, cnt, W_l1, b_l1, W_l2, b_l2)


# speedup vs baseline: 4.2199x; 1.6312x over previous
import functools

import jax
import jax.numpy as jnp
from jax import lax
from jax.experimental import pallas as pl
from jax.experimental.pallas import tpu as pltpu
from jax.experimental.pallas import tpu_sc as plsc

N, E, D, DE, H, G = 10000, 320000, 128, 16, 128, 64

_NC, _NS = 2, 16
_NW = _NC * _NS
_EPW = E // _NW
_C = 80
_NCHUNK = _EPW // _C
_NPAD = 10240
_RPT = _NPAD // _NS
_ZB = 16
_FR = 160

_BN_INV = 1.0 / (1.0 + 1e-5) ** 0.5


_sc_mesh = plsc.VectorSubcoreMesh(core_axis_name="c", subcore_axis_name="s")


_NB = 4


@functools.partial(
    pl.kernel,
    out_type=jax.ShapeDtypeStruct((_NC * _NPAD, D), jnp.float32),
    mesh=_sc_mesh,
    scratch_types=(
        [pltpu.VMEM((_C,), jnp.int32) for _ in range(_NB)]
        + [pltpu.VMEM((_C,), jnp.int32) for _ in range(_NB)]
        + [pltpu.VMEM((_C, D), jnp.float32) for _ in range(_NB)]
        + [
            pltpu.VMEM((_ZB, D), jnp.float32),
            pltpu.VMEM_SHARED((_NPAD, D), jnp.float32),
            pltpu.SemaphoreType.DMA,
            pltpu.SemaphoreType.DMA,
        ]
        + [pltpu.SemaphoreType.DMA for _ in range(_NB)]
    ),
)
def _sc_agg(x_hbm, e_hbm, src_hbm, dst_hbm, out_hbm, *scr):
    sidx = scr[:_NB]
    didx = scr[_NB:2 * _NB]
    bufs = scr[2 * _NB:3 * _NB]
    zbuf_v = scr[3 * _NB]
    agg_sh = scr[3 * _NB + 1]
    gs = scr[3 * _NB + 2]
    ss = scr[3 * _NB + 3]
    sls = scr[3 * _NB + 4:]
    c = lax.axis_index("c")
    s = lax.axis_index("s")
    w = c * _NS + s

    def _zrow(j, carry):
        for k in range(D // 16):
            zbuf_v[j, pl.ds(k * 16, 16)] = jnp.zeros((16,), jnp.float32)
        return carry

    lax.fori_loop(0, _ZB, _zrow, 0)
    for k in range(_RPT // _ZB):
        pltpu.async_copy(zbuf_v, agg_sh.at[pl.ds(s * _RPT + k * _ZB, _ZB)], gs)
    for k in range(_RPT // _ZB):
        pltpu.make_async_copy(
            zbuf_v, agg_sh.at[pl.ds(s * _RPT + k * _ZB, _ZB)], gs).wait()
    for _ in range(_C // _ZB):
        pltpu.async_copy(zbuf_v, agg_sh.at[pl.ds(s * _RPT, _ZB)], ss)

    def _issue_loads(jm, b):
        base = w * _EPW + jm * _C
        pltpu.async_copy(src_hbm.at[pl.ds(base, _C)], sidx[b], sls[b])
        pltpu.async_copy(dst_hbm.at[pl.ds(base, _C)], didx[b], sls[b])
        pltpu.async_copy(e_hbm.at[pl.ds(base, _C)], bufs[b], sls[b])

    def _wait_loads(b):
        pltpu.make_async_copy(src_hbm.at[pl.ds(0, _C)], sidx[b], sls[b]).wait()
        pltpu.make_async_copy(dst_hbm.at[pl.ds(0, _C)], didx[b], sls[b]).wait()
        pltpu.make_async_copy(e_hbm.at[pl.ds(0, _C)], bufs[b], sls[b]).wait()

    for j in range(_NB - 1):
        _issue_loads(j, j)
    _wait_loads(0)
    pltpu.async_copy(x_hbm.at[sidx[0]], bufs[0], gs, add=True)
    plsc.subcore_barrier()

    def _phase(i, ph):
        b = ph
        bn = (ph + 1) % _NB
        bl = (ph + _NB - 1) % _NB
        pltpu.make_async_copy(x_hbm.at[sidx[b]], bufs[b], gs).wait()

        def _rrow(j, cc):
            for k in range(D // 16):
                v = bufs[b][j, pl.ds(k * 16, 16)]
                bufs[b][j, pl.ds(k * 16, 16)] = jnp.maximum(v, 0.0)
            return cc

        lax.fori_loop(0, _C, _rrow, 0)
        pltpu.make_async_copy(bufs[bl], agg_sh.at[didx[bl]], ss).wait()
        _issue_loads(jnp.minimum(i + _NB - 1, _NCHUNK - 1), bl)
        pltpu.async_copy(bufs[b], agg_sh.at[didx[b]], ss, add=True)
        _wait_loads(bn)
        pltpu.async_copy(x_hbm.at[sidx[bn]], bufs[bn], gs, add=True)

    def _outer(k, carry):
        for ph in range(_NB):
            _phase(k * _NB + ph, ph)
        return carry

    lax.fori_loop(0, (_NCHUNK - 1) // _NB, _outer, 0)
    _phase(_NCHUNK - 1, (_NCHUNK - 1) % _NB)

    lastb = (_NCHUNK - 1) % _NB
    pltpu.make_async_copy(bufs[lastb], agg_sh.at[didx[lastb]], ss).wait()
    gb = (lastb + 1) % _NB
    pltpu.make_async_copy(x_hbm.at[sidx[gb]], bufs[gb], gs).wait()
    for b in ((lastb + 2) % _NB, (lastb + 3) % _NB):
        _wait_loads(b)
    plsc.subcore_barrier()

    for k in range(_RPT // _FR):
        r = s * _RPT + k * _FR
        pltpu.async_copy(agg_sh.at[pl.ds(r, _FR)],
                         out_hbm.at[pl.ds(c * _NPAD + r, _FR)], gs)
    for k in range(_RPT // _FR):
        r = s * _RPT + k * _FR
        pltpu.make_async_copy(agg_sh.at[pl.ds(r, _FR)],
                              out_hbm.at[pl.ds(c * _NPAD + r, _FR)], gs).wait()


_BE = 4000


def _edge_mm_body(ea_ref, We_ref, be_ref, out_ref):
    out_ref[...] = (
        jnp.dot(ea_ref[...], We_ref[...], preferred_element_type=jnp.float32)
        + be_ref[...]
    )


def _edge_mm(ea, We, be):
    return pl.pallas_call(
        _edge_mm_body,
        grid=(E // _BE,),
        in_specs=[
            pl.BlockSpec((_BE, DE), lambda i: (i, 0)),
            pl.BlockSpec((DE, H), lambda i: (0, 0)),
            pl.BlockSpec((1, H), lambda i: (0, 0)),
        ],
        out_specs=pl.BlockSpec((_BE, H), lambda i: (i, 0)),
        out_shape=jax.ShapeDtypeStruct((E, H), jnp.float32),
    )(ea, We, be.reshape(1, H))


_BNODE = 1000


def _layer_body(x_ref, a0_ref, a1_ref, b_ref, W1_ref, b1_ref, g_ref, bt_ref,
                W2_ref, b2_ref, h_ref, ps_ref, cnt_ref=None):
    i = pl.program_id(0)
    h = x_ref[...] + a0_ref[...] + a1_ref[...]
    t = jnp.dot(h, W1_ref[...], preferred_element_type=jnp.float32) + b1_ref[...]
    t = t * (_BN_INV * g_ref[...]) + bt_ref[...]
    t = jnp.maximum(t, 0.0)
    u = jnp.dot(t, W2_ref[...], preferred_element_type=jnp.float32) + b2_ref[...]
    u = jnp.maximum(u, 0.0)
    h_ref[...] = u
    onehot = (b_ref[...] == lax.broadcasted_iota(jnp.int32, (1, G), 1)
              ).astype(jnp.float32)
    ps = lax.dot_general(onehot, u, (((0,), (0,)), ((), ())),
                         preferred_element_type=jnp.float32)

    @pl.when(i == 0)
    def _():
        ps_ref[...] = ps

    @pl.when(i > 0)
    def _():
        ps_ref[...] += ps

    if cnt_ref is not None:
        cm = lax.dot_general(
            onehot, jnp.ones((_BNODE, H), jnp.float32),
            (((0,), (0,)), ((), ())), preferred_element_type=jnp.float32)

        @pl.when(i == 0)
        def _():
            cnt_ref[...] = cm

        @pl.when(i > 0)
        def _():
            cnt_ref[...] += cm


def _layer(x, a0, a1, batch2, W1, b1, g, bt, W2, b2, with_counts):
    body = _layer_body if with_counts else (
        lambda *refs: _layer_body(*refs, cnt_ref=None))
    row = lambda v: v.reshape(1, H)
    out_shape = [
        jax.ShapeDtypeStruct((N, H), jnp.float32),
        jax.ShapeDtypeStruct((G, H), jnp.float32),
    ]
    out_specs = [
        pl.BlockSpec((_BNODE, H), lambda i: (i, 0)),
        pl.BlockSpec((G, H), lambda i: (0, 0)),
    ]
    if with_counts:
        out_shape.append(jax.ShapeDtypeStruct((G, H), jnp.float32))
        out_specs.append(pl.BlockSpec((G, H), lambda i: (0, 0)))
    return pl.pallas_call(
        body,
        grid=(N // _BNODE,),
        in_specs=[
            pl.BlockSpec((_BNODE, H), lambda i: (i, 0)),
            pl.BlockSpec((_BNODE, H), lambda i: (i, 0)),
            pl.BlockSpec((_BNODE, H), lambda i: (i, 0)),
            pl.BlockSpec((_BNODE, 1), lambda i: (i, 0)),
            pl.BlockSpec((H, H), lambda i: (0, 0)),
            pl.BlockSpec((1, H), lambda i: (0, 0)),
            pl.BlockSpec((1, H), lambda i: (0, 0)),
            pl.BlockSpec((1, H), lambda i: (0, 0)),
            pl.BlockSpec((H, H), lambda i: (0, 0)),
            pl.BlockSpec((1, H), lambda i: (0, 0)),
        ],
        out_specs=out_specs,
        out_shape=out_shape,
    )(x, a0, a1, batch2, W1, row(b1), row(g), row(bt), W2, row(b2))


def _head_body(p1_ref, p2_ref, p3_ref, cnt_ref, W1_ref, b1_ref, W2_ref,
               b2_ref, out_ref):
    c = jnp.maximum(cnt_ref[...][:, 0:1], 1.0)
    q = jnp.concatenate(
        [p1_ref[...] / c, p2_ref[...] / c, p3_ref[...] / c], axis=1)
    t = jnp.dot(q, W1_ref[...], preferred_element_type=jnp.float32) + b1_ref[...]
    t = jnp.maximum(t, 0.0)
    z = jnp.dot(t, W2_ref[...], preferred_element_type=jnp.float32) + b2_ref[...]
    out_ref[...] = jax.nn.sigmoid(z)


def _head(p1, p2, p3, cnt, W1, b1, W2, b2):
    return pl.pallas_call(
        _head_body,
        out_shape=jax.ShapeDtypeStruct((G, 1), jnp.float32),
    )(p1, p2, p3, cnt, W1, b1.reshape(1, -1), W2, b2.reshape(1, -1))


def kernel(x, edge_index, edge_attr, batch,
           c1_We, c1_be, c1_W1, c1_b1, c1_g, c1_bt, c1_W2, c1_b2,
           c2_We, c2_be, c2_W1, c2_b1, c2_g, c2_bt, c2_W2, c2_b2,
           c3_We, c3_be, c3_W1, c3_b1, c3_g, c3_bt, c3_W2, c3_b2,
           W_l1, b_l1, W_l2, b_l2):
    src = edge_index[0]
    dst = edge_index[1]
    batch2 = batch.reshape(N, 1)

    e1 = _edge_mm(edge_attr, c1_We, c1_be)
    e2 = _edge_mm(edge_attr, c2_We, c2_be)
    e3 = _edge_mm(edge_attr, c3_We, c3_be)

    a1 = _sc_agg(x, e1, src, dst)
    h1, p1, cnt = _layer(x, a1[:N], a1[_NPAD:_NPAD + N], batch2,
                         c1_W1, c1_b1, c1_g, c1_bt, c1_W2, c1_b2, True)
    a2 = _sc_agg(h1, e2, src, dst)
    h2, p2 = _layer(h1, a2[:N], a2[_NPAD:_NPAD + N], batch2,
                    c2_W1, c2_b1, c2_g, c2_bt, c2_W2, c2_b2, False)
    a3 = _sc_agg(h2, e3, src, dst)
    h3, p3 = _layer(h2, a3[:N], a3[_NPAD:_NPAD + N], batch2,
                    c3_W1, c3_b1, c3_g, c3_bt, c3_W2, c3_b2, False)

    return _head(p1, p2, p3, cnt, W_l1, b_l1, W_l2, b_l2)


# speedup vs baseline: 4.7393x; 1.1231x over previous
import functools

import jax
import jax.numpy as jnp
from jax import lax
from jax.experimental import pallas as pl
from jax.experimental.pallas import tpu as pltpu
from jax.experimental.pallas import tpu_sc as plsc

N, E, D, DE, H, G = 10000, 320000, 128, 16, 128, 64

_NC, _NS = 2, 16
_NW = _NC * _NS
_EPW = E // _NW
_C = 80
_NCHUNK = _EPW // _C
_NPAD = 10240
_RPT = _NPAD // _NS
_ZB = 16
_FR = 160

_BN_INV = 1.0 / (1.0 + 1e-5) ** 0.5


_sc_mesh = plsc.VectorSubcoreMesh(core_axis_name="c", subcore_axis_name="s")


_NB = 4


@functools.partial(
    pl.kernel,
    out_type=jax.ShapeDtypeStruct((_NC * _NPAD, D), jnp.float32),
    mesh=_sc_mesh,
    scratch_types=(
        [pltpu.VMEM((_C,), jnp.int32) for _ in range(_NB)]
        + [pltpu.VMEM((_C,), jnp.int32) for _ in range(_NB)]
        + [pltpu.VMEM((_C, D), jnp.float32) for _ in range(_NB)]
        + [
            pltpu.VMEM((_ZB, D), jnp.float32),
            pltpu.VMEM_SHARED((_NPAD, D), jnp.float32),
            pltpu.SemaphoreType.DMA,
            pltpu.SemaphoreType.DMA,
        ]
        + [pltpu.SemaphoreType.DMA for _ in range(_NB)]
    ),
)
def _sc_agg(x_hbm, e_hbm, src_hbm, dst_hbm, out_hbm, *scr):
    sidx = scr[:_NB]
    didx = scr[_NB:2 * _NB]
    bufs = scr[2 * _NB:3 * _NB]
    zbuf_v = scr[3 * _NB]
    agg_sh = scr[3 * _NB + 1]
    gs = scr[3 * _NB + 2]
    ss = scr[3 * _NB + 3]
    sls = scr[3 * _NB + 4:]
    c = lax.axis_index("c")
    s = lax.axis_index("s")
    w = c * _NS + s

    def _zrow(j, carry):
        for k in range(D // 16):
            zbuf_v[j, pl.ds(k * 16, 16)] = jnp.zeros((16,), jnp.float32)
        return carry

    lax.fori_loop(0, _ZB, _zrow, 0)
    for k in range(_RPT // _ZB):
        pltpu.async_copy(zbuf_v, agg_sh.at[pl.ds(s * _RPT + k * _ZB, _ZB)], gs)
    for k in range(_RPT // _ZB):
        pltpu.make_async_copy(
            zbuf_v, agg_sh.at[pl.ds(s * _RPT + k * _ZB, _ZB)], gs).wait()
    for _ in range(_C // _ZB):
        pltpu.async_copy(zbuf_v, agg_sh.at[pl.ds(s * _RPT, _ZB)], ss)

    def _issue_loads(jm, b):
        base = w * _EPW + jm * _C
        pltpu.async_copy(src_hbm.at[pl.ds(base, _C)], sidx[b], sls[b])
        pltpu.async_copy(dst_hbm.at[pl.ds(base, _C)], didx[b], sls[b])
        pltpu.async_copy(e_hbm.at[pl.ds(base, _C)], bufs[b], sls[b])

    def _wait_loads(b):
        pltpu.make_async_copy(src_hbm.at[pl.ds(0, _C)], sidx[b], sls[b]).wait()
        pltpu.make_async_copy(dst_hbm.at[pl.ds(0, _C)], didx[b], sls[b]).wait()
        pltpu.make_async_copy(e_hbm.at[pl.ds(0, _C)], bufs[b], sls[b]).wait()

    for j in range(_NB - 1):
        _issue_loads(j, j)
    _wait_loads(0)
    pltpu.async_copy(x_hbm.at[sidx[0]], bufs[0], gs, add=True)
    plsc.subcore_barrier()

    def _phase(i, ph):
        b = ph
        bn = (ph + 1) % _NB
        bl = (ph + _NB - 1) % _NB
        pltpu.make_async_copy(x_hbm.at[sidx[b]], bufs[b], gs).wait()
        _wait_loads(bn)
        pltpu.async_copy(x_hbm.at[sidx[bn]], bufs[bn], gs, add=True)

        @plsc.parallel_loop(0, _C, unroll=4)
        def _rrow(j):
            for k in range(D // 16):
                v = bufs[b][j, pl.ds(k * 16, 16)]
                bufs[b][j, pl.ds(k * 16, 16)] = jnp.maximum(v, 0.0)

        pltpu.make_async_copy(bufs[bl], agg_sh.at[didx[bl]], ss).wait()
        _issue_loads(jnp.minimum(i + _NB - 1, _NCHUNK - 1), bl)
        pltpu.async_copy(bufs[b], agg_sh.at[didx[b]], ss, add=True)

    def _outer(k, carry):
        for ph in range(_NB):
            _phase(k * _NB + ph, ph)
        return carry

    lax.fori_loop(0, (_NCHUNK - 1) // _NB, _outer, 0)
    _phase(_NCHUNK - 1, (_NCHUNK - 1) % _NB)

    lastb = (_NCHUNK - 1) % _NB
    pltpu.make_async_copy(bufs[lastb], agg_sh.at[didx[lastb]], ss).wait()
    gb = (lastb + 1) % _NB
    pltpu.make_async_copy(x_hbm.at[sidx[gb]], bufs[gb], gs).wait()
    for b in ((lastb + 2) % _NB, (lastb + 3) % _NB):
        _wait_loads(b)
    plsc.subcore_barrier()

    for k in range(_RPT // _FR):
        r = s * _RPT + k * _FR
        pltpu.async_copy(agg_sh.at[pl.ds(r, _FR)],
                         out_hbm.at[pl.ds(c * _NPAD + r, _FR)], gs)
    for k in range(_RPT // _FR):
        r = s * _RPT + k * _FR
        pltpu.make_async_copy(agg_sh.at[pl.ds(r, _FR)],
                              out_hbm.at[pl.ds(c * _NPAD + r, _FR)], gs).wait()


_BE = 4000


def _edge_mm_body(ea_ref, We_ref, be_ref, out_ref):
    out_ref[...] = (
        jnp.dot(ea_ref[...], We_ref[...], preferred_element_type=jnp.float32)
        + be_ref[...]
    )


def _edge_mm(ea, We, be):
    return pl.pallas_call(
        _edge_mm_body,
        grid=(E // _BE,),
        in_specs=[
            pl.BlockSpec((_BE, DE), lambda i: (i, 0)),
            pl.BlockSpec((DE, H), lambda i: (0, 0)),
            pl.BlockSpec((1, H), lambda i: (0, 0)),
        ],
        out_specs=pl.BlockSpec((_BE, H), lambda i: (i, 0)),
        out_shape=jax.ShapeDtypeStruct((E, H), jnp.float32),
    )(ea, We, be.reshape(1, H))


_BNODE = 1000


def _layer_body(x_ref, a0_ref, a1_ref, b_ref, W1_ref, b1_ref, g_ref, bt_ref,
                W2_ref, b2_ref, h_ref, ps_ref, cnt_ref=None):
    i = pl.program_id(0)
    h = x_ref[...] + a0_ref[...] + a1_ref[...]
    t = jnp.dot(h, W1_ref[...], preferred_element_type=jnp.float32) + b1_ref[...]
    t = t * (_BN_INV * g_ref[...]) + bt_ref[...]
    t = jnp.maximum(t, 0.0)
    u = jnp.dot(t, W2_ref[...], preferred_element_type=jnp.float32) + b2_ref[...]
    u = jnp.maximum(u, 0.0)
    h_ref[...] = u
    onehot = (b_ref[...] == lax.broadcasted_iota(jnp.int32, (1, G), 1)
              ).astype(jnp.float32)
    ps = lax.dot_general(onehot, u, (((0,), (0,)), ((), ())),
                         preferred_element_type=jnp.float32)

    @pl.when(i == 0)
    def _():
        ps_ref[...] = ps

    @pl.when(i > 0)
    def _():
        ps_ref[...] += ps

    if cnt_ref is not None:
        cm = lax.dot_general(
            onehot, jnp.ones((_BNODE, H), jnp.float32),
            (((0,), (0,)), ((), ())), preferred_element_type=jnp.float32)

        @pl.when(i == 0)
        def _():
            cnt_ref[...] = cm

        @pl.when(i > 0)
        def _():
            cnt_ref[...] += cm


def _layer(x, a0, a1, batch2, W1, b1, g, bt, W2, b2, with_counts):
    body = _layer_body if with_counts else (
        lambda *refs: _layer_body(*refs, cnt_ref=None))
    row = lambda v: v.reshape(1, H)
    out_shape = [
        jax.ShapeDtypeStruct((N, H), jnp.float32),
        jax.ShapeDtypeStruct((G, H), jnp.float32),
    ]
    out_specs = [
        pl.BlockSpec((_BNODE, H), lambda i: (i, 0)),
        pl.BlockSpec((G, H), lambda i: (0, 0)),
    ]
    if with_counts:
        out_shape.append(jax.ShapeDtypeStruct((G, H), jnp.float32))
        out_specs.append(pl.BlockSpec((G, H), lambda i: (0, 0)))
    return pl.pallas_call(
        body,
        grid=(N // _BNODE,),
        in_specs=[
            pl.BlockSpec((_BNODE, H), lambda i: (i, 0)),
            pl.BlockSpec((_BNODE, H), lambda i: (i, 0)),
            pl.BlockSpec((_BNODE, H), lambda i: (i, 0)),
            pl.BlockSpec((_BNODE, 1), lambda i: (i, 0)),
            pl.BlockSpec((H, H), lambda i: (0, 0)),
            pl.BlockSpec((1, H), lambda i: (0, 0)),
            pl.BlockSpec((1, H), lambda i: (0, 0)),
            pl.BlockSpec((1, H), lambda i: (0, 0)),
            pl.BlockSpec((H, H), lambda i: (0, 0)),
            pl.BlockSpec((1, H), lambda i: (0, 0)),
        ],
        out_specs=out_specs,
        out_shape=out_shape,
    )(x, a0, a1, batch2, W1, row(b1), row(g), row(bt), W2, row(b2))


def _head_body(p1_ref, p2_ref, p3_ref, cnt_ref, W1_ref, b1_ref, W2_ref,
               b2_ref, out_ref):
    c = jnp.maximum(cnt_ref[...][:, 0:1], 1.0)
    q = jnp.concatenate(
        [p1_ref[...] / c, p2_ref[...] / c, p3_ref[...] / c], axis=1)
    t = jnp.dot(q, W1_ref[...], preferred_element_type=jnp.float32) + b1_ref[...]
    t = jnp.maximum(t, 0.0)
    z = jnp.dot(t, W2_ref[...], preferred_element_type=jnp.float32) + b2_ref[...]
    out_ref[...] = jax.nn.sigmoid(z)


def _head(p1, p2, p3, cnt, W1, b1, W2, b2):
    return pl.pallas_call(
        _head_body,
        out_shape=jax.ShapeDtypeStruct((G, 1), jnp.float32),
    )(p1, p2, p3, cnt, W1, b1.reshape(1, -1), W2, b2.reshape(1, -1))


def kernel(x, edge_index, edge_attr, batch,
           c1_We, c1_be, c1_W1, c1_b1, c1_g, c1_bt, c1_W2, c1_b2,
           c2_We, c2_be, c2_W1, c2_b1, c2_g, c2_bt, c2_W2, c2_b2,
           c3_We, c3_be, c3_W1, c3_b1, c3_g, c3_bt, c3_W2, c3_b2,
           W_l1, b_l1, W_l2, b_l2):
    src = edge_index[0]
    dst = edge_index[1]
    batch2 = batch.reshape(N, 1)

    e1 = _edge_mm(edge_attr, c1_We, c1_be)
    e2 = _edge_mm(edge_attr, c2_We, c2_be)
    e3 = _edge_mm(edge_attr, c3_We, c3_be)

    a1 = _sc_agg(x, e1, src, dst)
    h1, p1, cnt = _layer(x, a1[:N], a1[_NPAD:_NPAD + N], batch2,
                         c1_W1, c1_b1, c1_g, c1_bt, c1_W2, c1_b2, True)
    a2 = _sc_agg(h1, e2, src, dst)
    h2, p2 = _layer(h1, a2[:N], a2[_NPAD:_NPAD + N], batch2,
                    c2_W1, c2_b1, c2_g, c2_bt, c2_W2, c2_b2, False)
    a3 = _sc_agg(h2, e3, src, dst)
    h3, p3 = _layer(h2, a3[:N], a3[_NPAD:_NPAD + N], batch2,
                    c3_W1, c3_b1, c3_g, c3_bt, c3_W2, c3_b2, False)

    return _head(p1, p2, p3, cnt, W_l1, b_l1, W_l2, b_l2)
